# baseline (speedup 1.0000x reference)
import jax
import jax.numpy as jnp
from jax import lax
from jax.experimental import pallas as pl
from jax.experimental.pallas import tpu as pltpu
from jax.experimental.pallas import tpu_sc as plsc

_K = 128
_NW = 32


def _bc_i32(v):
    return jax.lax.bitcast_convert_type(v, jnp.int32)


def _bc_f32(v):
    return jax.lax.bitcast_convert_type(v, jnp.float32)


def _enc_kernel(x_ref, w_ref, b_ref, out_ref):
    acc = jax.lax.dot_general(
        x_ref[:], w_ref[:], (((1,), (1,)), ((), ())),
        preferred_element_type=jnp.float32)
    out_ref[:] = jnp.maximum(acc + b_ref[:], 0.0)


def _thresh_kernel(u_ref, t_ref):
    rows, cols = u_ref.shape
    m = u_ref[:]
    s = cols // 2
    while s >= 128:
        m = jnp.maximum(m[:, :s], m[:, s:])
        s //= 2
    lo0 = _bc_i32(jnp.min(m, axis=1, keepdims=True))
    hi0 = _bc_i32(jnp.max(m, axis=1, keepdims=True)) + 1
    found0 = jnp.zeros((rows, 1), jnp.int32)
    ts0 = jnp.zeros((rows, 1), jnp.int32)

    def cond(c):
        lo, hi, found, _ = c
        return jnp.max((hi - lo) * (1 - found)) > 1

    def body(c):
        lo, hi, found, ts = c
        mid = lo + ((hi - lo) >> 1)
        midf = _bc_f32(mid)
        cnt = jnp.sum((u_ref[:] >= midf).astype(jnp.float32),
                      axis=1, keepdims=True)
        exact = jnp.where(cnt == float(_K), 1, 0)
        ts = jnp.where(exact * (1 - found) == 1, mid, ts)
        found = jnp.maximum(found, exact)
        pred = cnt >= float(_K)
        lo = jnp.where(pred, mid, lo)
        hi = jnp.where(pred, hi, mid)
        return lo, hi, found, ts

    lo, _, found, ts = jax.lax.while_loop(cond, body, (lo0, hi0, found0, ts0))
    t_ref[:] = _bc_f32(jnp.where(found == 1, ts, lo))


def _sc_thresh_kernel(u_hbm, t_hbm, row_v, t_v, hist_v, sem):
    cols = u_hbm.shape[1]
    rpw = u_hbm.shape[0] // _NW
    nck = cols // 16
    c = lax.axis_index("c")
    s = lax.axis_index("s")
    wid = s * 2 + c
    base = wid * rpw

    idx16 = lax.iota(jnp.int32, 16)
    ones16 = jnp.ones((16,), jnp.int32)
    zeros16 = jnp.zeros((16,), jnp.int32)

    def clear_hist(nbuckets):
        def clr(i, carry):
            hist_v[pl.ds(i * 16, 16)] = zeros16
            return carry
        lax.fori_loop(0, nbuckets // 16, clr, 0)

    def scan_hist(start_chunk, k):
        def cond(cr):
            j, _, _, _, found = cr
            return jnp.logical_and(found == 0, j >= 0)

        def body(cr):
            j, acc, bkt, above, found = cr
            h = hist_v[pl.ds(j * 16, 16)]
            tot = jnp.sum(h)
            hit = (acc + tot) >= k
            sfx = lax.rev(jnp.cumsum(lax.rev(h, (0,)), axis=0), (0,))
            m = (acc + sfx) >= k
            bl = jnp.maximum(jnp.max(jnp.where(m, idx16, -1)), 0)
            sfx_at = jnp.max(jnp.where(idx16 == bl, sfx, 0))
            h_at = jnp.max(jnp.where(idx16 == bl, h, 0))
            bkt = jnp.where(hit, j * 16 + bl, bkt)
            above = jnp.where(hit, acc + sfx_at - h_at, above)
            return (j - 1, acc + tot, bkt, above,
                    jnp.where(hit, 1, found))

        _, _, bkt, above, _ = lax.while_loop(
            cond, body, (start_chunk, 0, 0, 0, 0))
        return bkt, above

    pltpu.make_async_copy(u_hbm.at[base], row_v.at[pl.ds(0, cols)],
                          sem).start()

    def row_fn(r, carry):
        off = lax.rem(r, 2) * cols
        pltpu.make_async_copy(u_hbm.at[base + r], row_v.at[pl.ds(off, cols)],
                              sem).wait()

        @pl.when(r < rpw - 1)
        def _():
            noff = lax.rem(r + 1, 2) * cols
            pltpu.make_async_copy(u_hbm.at[base + r + 1],
                                  row_v.at[pl.ds(noff, cols)], sem).start()

        clear_hist(2048)

        def p1(i, vm):
            v = row_v[pl.ds(off + i * 16, 16)]
            b = lax.shift_right_logical(v, 20)
            plsc.addupdate_scatter(hist_v, [b], ones16)
            return jnp.maximum(vm, v)

        vm = lax.fori_loop(0, nck, p1, jnp.zeros((16,), jnp.int32))
        maxb = lax.shift_right_logical(jnp.max(vm), 20)
        b1, ab1 = scan_hist(maxb // 16, _K)
        k2 = _K - ab1

        clear_hist(2048)

        def p2(i, carry):
            bits = row_v[pl.ds(off + i * 16, 16)]
            msk = lax.shift_right_logical(bits, 20) == b1
            key = jnp.bitwise_and(lax.shift_right_logical(bits, 9), 0x7FF)
            plsc.addupdate_scatter(hist_v, [key], ones16, mask=msk)
            return carry

        lax.fori_loop(0, nck, p2, 0)
        b2, ab2 = scan_hist(127, k2)
        k3 = k2 - ab2

        clear_hist(512)

        def p3(i, carry):
            bits = row_v[pl.ds(off + i * 16, 16)]
            msk = jnp.logical_and(
                lax.shift_right_logical(bits, 20) == b1,
                jnp.bitwise_and(lax.shift_right_logical(bits, 9), 0x7FF) == b2)
            key = jnp.bitwise_and(bits, 0x1FF)
            plsc.addupdate_scatter(hist_v, [key], ones16, mask=msk)
            return carry

        lax.fori_loop(0, nck, p3, 0)
        b3, _ = scan_hist(31, k3)

        tbits = jnp.bitwise_or(
            jnp.bitwise_or(lax.shift_left(b1, 20), lax.shift_left(b2, 9)), b3)
        return jnp.where(idx16 == lax.rem(r, 16), tbits, carry)

    def grp_fn(g, carry):
        tvec = lax.fori_loop(g * 16, g * 16 + 16, row_fn,
                             jnp.zeros((16,), jnp.int32))
        t_v[pl.ds(g * 16, 16)] = tvec
        return carry

    lax.fori_loop(0, rpw // 16, grp_fn, 0)
    pltpu.sync_copy(t_v, t_hbm.at[pl.ds(base, rpw)])


def _dec_kernel(u_ref, t_ref, w_ref, sparse_ref, recon_ref):
    j = pl.program_id(1)
    u = u_ref[:]
    sparse = jnp.where(u >= t_ref[:], u, 0.0)
    sparse_ref[:] = sparse
    contrib = jax.lax.dot_general(
        sparse, w_ref[:], (((1,), (1,)), ((), ())),
        preferred_element_type=jnp.float32)

    @pl.when(j == 0)
    def _():
        recon_ref[:] = jnp.zeros_like(recon_ref)

    recon_ref[:] += contrib


def kernel(x, W_enc, b_enc, W_dec):
    n, d = x.shape
    dict_size = W_enc.shape[0]

    bm_a = min(2048, n)
    bn_a = min(512, dict_size)
    rb = min(128, n)
    rc = min(1024, n)
    bn_c = min(1024, dict_size)

    b2 = b_enc.reshape(1, dict_size)

    u = pl.pallas_call(
        _enc_kernel,
        grid=(n // bm_a, dict_size // bn_a),
        in_specs=[
            pl.BlockSpec((bm_a, d), lambda i, j: (i, 0)),
            pl.BlockSpec((bn_a, d), lambda i, j: (j, 0)),
            pl.BlockSpec((1, bn_a), lambda i, j: (0, j)),
        ],
        out_specs=pl.BlockSpec((bm_a, bn_a), lambda i, j: (i, j)),
        out_shape=jax.ShapeDtypeStruct((n, dict_size), jnp.float32),
    )(x, W_enc, b2)

    u_bits = _bc_i32(u)
    t_flat = pl.kernel(
        _sc_thresh_kernel,
        out_type=jax.ShapeDtypeStruct((n,), jnp.int32),
        mesh=plsc.VectorSubcoreMesh(core_axis_name="c", subcore_axis_name="s"),
        compiler_params=pltpu.CompilerParams(needs_layout_passes=False),
        scratch_types=[
            pltpu.VMEM((2 * dict_size,), jnp.int32),
            pltpu.VMEM((n // _NW,), jnp.int32),
            pltpu.VMEM((2048,), jnp.int32),
            pltpu.SemaphoreType.DMA,
        ],
    )(u_bits)
    t = _bc_f32(t_flat).reshape(n, 1)

    sparse, recon = pl.pallas_call(
        _dec_kernel,
        grid=(n // rc, dict_size // bn_c),
        in_specs=[
            pl.BlockSpec((rc, bn_c), lambda i, j: (i, j)),
            pl.BlockSpec((rc, 1), lambda i, j: (i, 0)),
            pl.BlockSpec((d, bn_c), lambda i, j: (0, j)),
        ],
        out_specs=[
            pl.BlockSpec((rc, bn_c), lambda i, j: (i, j)),
            pl.BlockSpec((rc, d), lambda i, j: (i, 0)),
        ],
        out_shape=[
            jax.ShapeDtypeStruct((n, dict_size), jnp.float32),
            jax.ShapeDtypeStruct((n, d), jnp.float32),
        ],
        compiler_params=pltpu.CompilerParams(
            dimension_semantics=("arbitrary", "arbitrary")),
    )(u, t, W_dec)

    return recon, sparse


# speedup vs baseline: 1.1235x; 1.1235x over previous
import jax
import jax.numpy as jnp
from jax import lax
from jax.experimental import pallas as pl
from jax.experimental.pallas import tpu as pltpu
from jax.experimental.pallas import tpu_sc as plsc

_K = 128
_NW = 32


def _bc_i32(v):
    return jax.lax.bitcast_convert_type(v, jnp.int32)


def _bc_f32(v):
    return jax.lax.bitcast_convert_type(v, jnp.float32)


def _enc_kernel(x_ref, w_ref, b_ref, out_ref):
    acc = jax.lax.dot_general(
        x_ref[:], w_ref[:], (((1,), (1,)), ((), ())),
        preferred_element_type=jnp.float32)
    out_ref[:] = jnp.maximum(acc + b_ref[:], 0.0)


def _thresh_kernel(u_ref, t_ref):
    rows, cols = u_ref.shape
    m = u_ref[:]
    s = cols // 2
    while s >= 128:
        m = jnp.maximum(m[:, :s], m[:, s:])
        s //= 2
    lo0 = _bc_i32(jnp.min(m, axis=1, keepdims=True))
    hi0 = _bc_i32(jnp.max(m, axis=1, keepdims=True)) + 1
    found0 = jnp.zeros((rows, 1), jnp.int32)
    ts0 = jnp.zeros((rows, 1), jnp.int32)

    def cond(c):
        lo, hi, found, _ = c
        return jnp.max((hi - lo) * (1 - found)) > 1

    def body(c):
        lo, hi, found, ts = c
        mid = lo + ((hi - lo) >> 1)
        midf = _bc_f32(mid)
        cnt = jnp.sum((u_ref[:] >= midf).astype(jnp.float32),
                      axis=1, keepdims=True)
        exact = jnp.where(cnt == float(_K), 1, 0)
        ts = jnp.where(exact * (1 - found) == 1, mid, ts)
        found = jnp.maximum(found, exact)
        pred = cnt >= float(_K)
        lo = jnp.where(pred, mid, lo)
        hi = jnp.where(pred, hi, mid)
        return lo, hi, found, ts

    lo, _, found, ts = jax.lax.while_loop(cond, body, (lo0, hi0, found0, ts0))
    t_ref[:] = _bc_f32(jnp.where(found == 1, ts, lo))


def _sc_thresh_kernel(u_hbm, t_hbm, row_v, t_v, hist_v, sem):
    cols = u_hbm.shape[1]
    rpw = u_hbm.shape[0] // _NW
    nck = cols // 16
    c = lax.axis_index("c")
    s = lax.axis_index("s")
    wid = s * 2 + c
    base = wid * rpw

    idx16 = lax.iota(jnp.int32, 16)
    ones16 = jnp.ones((16,), jnp.int32)
    zeros16 = jnp.zeros((16,), jnp.int32)

    def clear_hist(nbuckets):
        def clr(i, carry):
            hist_v[pl.ds(i * 16, 16)] = zeros16
            return carry
        lax.fori_loop(0, nbuckets // 16, clr, 0)

    def scan_hist(start_chunk, k):
        def cond(cr):
            j, _, _, _, found = cr
            return jnp.logical_and(found == 0, j >= 0)

        def body(cr):
            j, acc, bkt, above, found = cr
            h = hist_v[pl.ds(j * 16, 16)]
            tot = jnp.sum(h)
            hit = (acc + tot) >= k
            sfx = lax.rev(jnp.cumsum(lax.rev(h, (0,)), axis=0), (0,))
            m = (acc + sfx) >= k
            bl = jnp.maximum(jnp.max(jnp.where(m, idx16, -1)), 0)
            sfx_at = jnp.max(jnp.where(idx16 == bl, sfx, 0))
            h_at = jnp.max(jnp.where(idx16 == bl, h, 0))
            bkt = jnp.where(hit, j * 16 + bl, bkt)
            above = jnp.where(hit, acc + sfx_at - h_at, above)
            return (j - 1, acc + tot, bkt, above,
                    jnp.where(hit, 1, found))

        _, _, bkt, above, _ = lax.while_loop(
            cond, body, (start_chunk, 0, 0, 0, 0))
        return bkt, above

    pltpu.make_async_copy(u_hbm.at[base], row_v.at[pl.ds(0, cols)],
                          sem).start()

    def row_fn(r, carry):
        off = lax.rem(r, 2) * cols
        pltpu.make_async_copy(u_hbm.at[base + r], row_v.at[pl.ds(off, cols)],
                              sem).wait()

        @pl.when(r < rpw - 1)
        def _():
            noff = lax.rem(r + 1, 2) * cols
            pltpu.make_async_copy(u_hbm.at[base + r + 1],
                                  row_v.at[pl.ds(noff, cols)], sem).start()

        clear_hist(2048)

        def p1(i, vm):
            v = row_v[pl.ds(off + i * 16, 16)]
            b = lax.shift_right_logical(v, 20)
            plsc.addupdate_scatter(hist_v, [b], ones16, mask=v > 0)
            return jnp.maximum(vm, v)

        vm = lax.fori_loop(0, nck, p1, jnp.zeros((16,), jnp.int32),
                           unroll=8)
        maxb = lax.shift_right_logical(jnp.max(vm), 20)
        b1, ab1 = scan_hist(maxb // 16, _K)
        k2 = _K - ab1

        clear_hist(2048)

        def p2(i, carry):
            bits = row_v[pl.ds(off + i * 16, 16)]
            msk = lax.shift_right_logical(bits, 20) == b1
            key = jnp.bitwise_and(lax.shift_right_logical(bits, 9), 0x7FF)
            plsc.addupdate_scatter(hist_v, [key], ones16, mask=msk)
            return carry

        lax.fori_loop(0, nck, p2, 0, unroll=8)
        b2, ab2 = scan_hist(127, k2)
        k3 = k2 - ab2

        clear_hist(512)

        def p3(i, carry):
            bits = row_v[pl.ds(off + i * 16, 16)]
            msk = jnp.logical_and(
                lax.shift_right_logical(bits, 20) == b1,
                jnp.bitwise_and(lax.shift_right_logical(bits, 9), 0x7FF) == b2)
            key = jnp.bitwise_and(bits, 0x1FF)
            plsc.addupdate_scatter(hist_v, [key], ones16, mask=msk)
            return carry

        lax.fori_loop(0, nck, p3, 0, unroll=8)
        b3, _ = scan_hist(31, k3)

        tbits = jnp.bitwise_or(
            jnp.bitwise_or(lax.shift_left(b1, 20), lax.shift_left(b2, 9)), b3)
        return jnp.where(idx16 == lax.rem(r, 16), tbits, carry)

    def grp_fn(g, carry):
        tvec = lax.fori_loop(g * 16, g * 16 + 16, row_fn,
                             jnp.zeros((16,), jnp.int32))
        t_v[pl.ds(g * 16, 16)] = tvec
        return carry

    lax.fori_loop(0, rpw // 16, grp_fn, 0)
    pltpu.sync_copy(t_v, t_hbm.at[pl.ds(base, rpw)])


def _dec_kernel(u_ref, t_ref, w_ref, sparse_ref, recon_ref):
    j = pl.program_id(1)
    u = u_ref[:]
    sparse = jnp.where(u >= t_ref[:], u, 0.0)
    sparse_ref[:] = sparse
    contrib = jax.lax.dot_general(
        sparse, w_ref[:], (((1,), (1,)), ((), ())),
        preferred_element_type=jnp.float32)

    @pl.when(j == 0)
    def _():
        recon_ref[:] = jnp.zeros_like(recon_ref)

    recon_ref[:] += contrib


def kernel(x, W_enc, b_enc, W_dec):
    n, d = x.shape
    dict_size = W_enc.shape[0]

    bm_a = min(2048, n)
    bn_a = min(512, dict_size)
    rb = min(128, n)
    rc = min(1024, n)
    bn_c = min(1024, dict_size)

    b2 = b_enc.reshape(1, dict_size)

    u = pl.pallas_call(
        _enc_kernel,
        grid=(n // bm_a, dict_size // bn_a),
        in_specs=[
            pl.BlockSpec((bm_a, d), lambda i, j: (i, 0)),
            pl.BlockSpec((bn_a, d), lambda i, j: (j, 0)),
            pl.BlockSpec((1, bn_a), lambda i, j: (0, j)),
        ],
        out_specs=pl.BlockSpec((bm_a, bn_a), lambda i, j: (i, j)),
        out_shape=jax.ShapeDtypeStruct((n, dict_size), jnp.float32),
    )(x, W_enc, b2)

    u_bits = _bc_i32(u)
    t_flat = pl.kernel(
        _sc_thresh_kernel,
        out_type=jax.ShapeDtypeStruct((n,), jnp.int32),
        mesh=plsc.VectorSubcoreMesh(core_axis_name="c", subcore_axis_name="s"),
        compiler_params=pltpu.CompilerParams(needs_layout_passes=False),
        scratch_types=[
            pltpu.VMEM((2 * dict_size,), jnp.int32),
            pltpu.VMEM((n // _NW,), jnp.int32),
            pltpu.VMEM((2048,), jnp.int32),
            pltpu.SemaphoreType.DMA,
        ],
    )(u_bits)
    t = _bc_f32(t_flat).reshape(n, 1)

    sparse, recon = pl.pallas_call(
        _dec_kernel,
        grid=(n // rc, dict_size // bn_c),
        in_specs=[
            pl.BlockSpec((rc, bn_c), lambda i, j: (i, j)),
            pl.BlockSpec((rc, 1), lambda i, j: (i, 0)),
            pl.BlockSpec((d, bn_c), lambda i, j: (0, j)),
        ],
        out_specs=[
            pl.BlockSpec((rc, bn_c), lambda i, j: (i, j)),
            pl.BlockSpec((rc, d), lambda i, j: (i, 0)),
        ],
        out_shape=[
            jax.ShapeDtypeStruct((n, dict_size), jnp.float32),
            jax.ShapeDtypeStruct((n, d), jnp.float32),
        ],
        compiler_params=pltpu.CompilerParams(
            dimension_semantics=("arbitrary", "arbitrary")),
    )(u, t, W_dec)

    return recon, sparse
